# baseline (speedup 1.0000x reference)
import functools

import jax
import jax.numpy as jnp
from jax import lax
from jax.experimental import pallas as pl
from jax.experimental.pallas import tpu as pltpu
from jax.experimental.pallas import tpu_sc as plsc

N = 10000
E = 320000
D = 128

NC = 2
NS = 16
NW = NC * NS
EPW = E // NW
CK = 40
NCH = EPW // CK
NP = 10240
RPT = NP // NS
NDMP = RPT // CK

_MESH = plsc.VectorSubcoreMesh(core_axis_name="c", subcore_axis_name="s")


@functools.partial(
    pl.kernel,
    out_type=jax.ShapeDtypeStruct((NC, NP), jnp.float32),
    mesh=_MESH,
    scratch_types=[
        pltpu.VMEM_SHARED((NP,), jnp.float32),
        pltpu.VMEM((EPW,), jnp.int32),
        pltpu.VMEM((CK,), jnp.float32),
        pltpu.VMEM((RPT,), jnp.float32),
        pltpu.SemaphoreType.DMA,
    ],
)
def _deg_kernel(dst_hbm, deg_out, acc_sh, didx_v, ones_v, tmp_v, sem):
    cid = lax.axis_index("c")
    sid = lax.axis_index("s")
    w = cid * NS + sid

    zeros16 = jnp.zeros((16,), jnp.float32)
    ones16 = jnp.ones((16,), jnp.float32)

    def zstage(k, carry):
        tmp_v[pl.ds(k * 16, 16)] = zeros16
        return carry

    lax.fori_loop(0, RPT // 16, zstage, 0)

    ones_v[pl.ds(0, 16)] = ones16
    ones_v[pl.ds(16, 16)] = ones16
    ones_v[pl.ds(CK - 16, 16)] = ones16

    pltpu.sync_copy(tmp_v, acc_sh.at[pl.ds(sid * RPT, RPT)])
    pltpu.sync_copy(dst_hbm.at[pl.ds(w * EPW, EPW)], didx_v)
    plsc.subcore_barrier()

    def wave(g, carry):
        def fire(i, c):
            pltpu.async_copy(
                ones_v, acc_sh.at[didx_v.at[pl.ds((g * 25 + i) * CK, CK)]],
                sem, add=True)
            return c

        lax.fori_loop(0, 25, fire, 0)

        def drain(i, c):
            pltpu.make_async_copy(
                dst_hbm.at[pl.ds(0, CK)], didx_v.at[pl.ds(0, CK)], sem).wait()
            return c

        lax.fori_loop(0, 25, drain, 0)
        return carry

    lax.fori_loop(0, NCH // 25, wave, 0)
    plsc.subcore_barrier()

    pltpu.sync_copy(acc_sh.at[pl.ds(sid * RPT, RPT)], tmp_v)
    pltpu.sync_copy(tmp_v, deg_out.at[cid, pl.ds(sid * RPT, RPT)])


@functools.partial(
    pl.kernel,
    out_type=jax.ShapeDtypeStruct((NC, NP, D), jnp.float32),
    mesh=_MESH,
    scratch_types=[
        pltpu.VMEM_SHARED((NP, D), jnp.float32),
        pltpu.VMEM((EPW,), jnp.int32),
        pltpu.VMEM((EPW,), jnp.int32),
        [pltpu.VMEM((CK, D), jnp.float32) for _ in range(5)],
        pltpu.SemaphoreType.DMA((5,)),
        pltpu.SemaphoreType.DMA((5,)),
    ],
)
def _agg_kernel(g_hbm, src_hbm, dst_hbm, part_out,
                acc_sh, sidx_v, didx_v, rows, gsem, ssem):
    NR = 5
    NG = NCH // NR
    cid = lax.axis_index("c")
    sid = lax.axis_index("s")
    w = cid * NS + sid

    zeros16 = jnp.zeros((16,), jnp.float32)

    def ztmp(k, carry):
        rows[0][k // 8, pl.ds((k % 8) * 16, 16)] = zeros16
        return carry

    lax.fori_loop(0, CK * (D // 16), ztmp, 0)

    def zacc(k, carry):
        pltpu.async_copy(rows[0], acc_sh.at[pl.ds(sid * RPT + k * CK, CK), :],
                         gsem.at[0])
        return carry

    lax.fori_loop(0, NDMP, zacc, 0)
    pltpu.sync_copy(src_hbm.at[pl.ds(w * EPW, EPW)], sidx_v)
    pltpu.sync_copy(dst_hbm.at[pl.ds(w * EPW, EPW)], didx_v)

    def zdrain(k, carry):
        pltpu.make_async_copy(g_hbm.at[pl.ds(0, CK)], rows[0], gsem.at[0]).wait()
        return carry

    lax.fori_loop(0, NDMP, zdrain, 0)
    plsc.subcore_barrier()

    def emit_group(g, drain_scatter):
        gcps = []
        for b in range(NR):
            i = g * NR + b
            if drain_scatter:
                pltpu.make_async_copy(g_hbm.at[pl.ds(0, CK)], rows[b],
                                      ssem.at[b]).wait()
            gcps.append(pltpu.async_copy(
                g_hbm.at[sidx_v.at[pl.ds(i * CK, CK)]], rows[b], gsem.at[b]))
        for b in range(NR):
            i = g * NR + b
            gcps[b].wait()
            pltpu.async_copy(rows[b], acc_sh.at[didx_v.at[pl.ds(i * CK, CK)]],
                             ssem.at[b], add=True)

    emit_group(0, False)

    def group(g, carry):
        emit_group(g, True)
        return carry

    lax.fori_loop(1, NG, group, 0)
    for b in range(NR):
        pltpu.make_async_copy(g_hbm.at[pl.ds(0, CK)], rows[b], ssem.at[b]).wait()
    plsc.subcore_barrier()

    def demit(k, drain):
        for b in range(4):
            r0 = sid * RPT + (k * 4 + b) * CK
            if drain:
                pltpu.make_async_copy(g_hbm.at[pl.ds(0, CK)], rows[b],
                                      ssem.at[b]).wait()
            pltpu.async_copy(acc_sh.at[pl.ds(r0, CK), :], rows[b], gsem.at[b])
        for b in range(4):
            r0 = sid * RPT + (k * 4 + b) * CK
            pltpu.make_async_copy(g_hbm.at[pl.ds(0, CK)], rows[b],
                                  gsem.at[b]).wait()
            pltpu.async_copy(rows[b], part_out.at[cid, pl.ds(r0, CK), :],
                             ssem.at[b])

    demit(0, False)

    def dgroup(k, carry):
        demit(k, True)
        return carry

    lax.fori_loop(1, NDMP // 4, dgroup, 0)
    for b in range(4):
        pltpu.make_async_copy(g_hbm.at[pl.ds(0, CK)], rows[b], ssem.at[b]).wait()


BN = 2000


def _tcm_body(x, w1, h1):
    h1[...] = jnp.dot(x[...], w1[...], preferred_element_type=jnp.float32)


_tcm = pl.pallas_call(
    _tcm_body,
    grid=(N // BN,),
    in_specs=[
        pl.BlockSpec((BN, D), lambda i: (i, 0)),
        pl.BlockSpec((D, D), lambda i: (0, 0)),
    ],
    out_specs=pl.BlockSpec((BN, D), lambda i: (i, 0)),
    out_shape=jax.ShapeDtypeStruct((N, D), jnp.float32),
)


def _tca_body(degp, h, g1, dinv):
    deg = degp[0] + degp[1] + 1.0
    di = lax.rsqrt(deg)
    g1[...] = h[...] * di
    dinv[...] = di


_tca = pl.pallas_call(
    _tca_body,
    grid=(N // BN,),
    in_specs=[
        pl.BlockSpec((NC, BN, 1), lambda i: (0, i, 0)),
        pl.BlockSpec((BN, D), lambda i: (i, 0)),
    ],
    out_specs=[
        pl.BlockSpec((BN, D), lambda i: (i, 0)),
        pl.BlockSpec((BN, 1), lambda i: (i, 0)),
    ],
    out_shape=[
        jax.ShapeDtypeStruct((N, D), jnp.float32),
        jax.ShapeDtypeStruct((N, 1), jnp.float32),
    ],
)


def _tcb_body(p, g1, dinv, b1, w2, g2):
    agg = p[0] + p[1] + g1[...]
    di = dinv[...]
    out1 = jnp.maximum(agg * di + b1[...], 0.0)
    h2 = jnp.dot(out1, w2[...], preferred_element_type=jnp.float32)
    g2[...] = h2 * di


_tcb = pl.pallas_call(
    _tcb_body,
    grid=(N // BN,),
    in_specs=[
        pl.BlockSpec((NC, BN, D), lambda i: (0, i, 0)),
        pl.BlockSpec((BN, D), lambda i: (i, 0)),
        pl.BlockSpec((BN, 1), lambda i: (i, 0)),
        pl.BlockSpec((1, D), lambda i: (0, 0)),
        pl.BlockSpec((D, D), lambda i: (0, 0)),
    ],
    out_specs=pl.BlockSpec((BN, D), lambda i: (i, 0)),
    out_shape=jax.ShapeDtypeStruct((N, D), jnp.float32),
)


def _tcc_body(p, g2, dinv, b2, out):
    out[...] = (p[0] + p[1] + g2[...]) * dinv[...] + b2[...]


_tcc = pl.pallas_call(
    _tcc_body,
    grid=(N // BN,),
    in_specs=[
        pl.BlockSpec((NC, BN, D), lambda i: (0, i, 0)),
        pl.BlockSpec((BN, D), lambda i: (i, 0)),
        pl.BlockSpec((BN, 1), lambda i: (i, 0)),
        pl.BlockSpec((1, D), lambda i: (0, 0)),
    ],
    out_specs=pl.BlockSpec((BN, D), lambda i: (i, 0)),
    out_shape=jax.ShapeDtypeStruct((N, D), jnp.float32),
)


def kernel(x, edge_index, W1, b1, W2, b2):
    ei = edge_index.astype(jnp.int32)
    src = ei[0]
    dst = ei[1]
    b1r = b1.reshape(1, D)
    b2r = b2.reshape(1, D)

    degp = _deg_kernel(dst).reshape(NC, NP, 1)
    h1 = _tcm(x, W1)
    g1, dinv = _tca(degp, h1)
    part1 = _agg_kernel(g1, src, dst)
    g2 = _tcb(part1, g1, dinv, b1r, W2)
    part2 = _agg_kernel(g2, src, dst)
    out = _tcc(part2, g2, dinv, b2r)
    return out


# speedup vs baseline: 1.0031x; 1.0031x over previous
import functools

import jax
import jax.numpy as jnp
from jax import lax
from jax.experimental import pallas as pl
from jax.experimental.pallas import tpu as pltpu
from jax.experimental.pallas import tpu_sc as plsc

N = 10000
E = 320000
D = 128

NC = 2
NS = 16
NW = NC * NS
EPW = E // NW
CK = 40
NCH = EPW // CK
NP = 10240
RPT = NP // NS
NDMP = RPT // CK

_MESH = plsc.VectorSubcoreMesh(core_axis_name="c", subcore_axis_name="s")


@functools.partial(
    pl.kernel,
    out_type=jax.ShapeDtypeStruct((NC, NP), jnp.float32),
    mesh=_MESH,
    scratch_types=[
        pltpu.VMEM_SHARED((NP,), jnp.float32),
        pltpu.VMEM((EPW,), jnp.int32),
        pltpu.VMEM((CK,), jnp.float32),
        pltpu.VMEM((RPT,), jnp.float32),
        pltpu.SemaphoreType.DMA,
    ],
)
def _deg_kernel(dst_hbm, deg_out, acc_sh, didx_v, ones_v, tmp_v, sem):
    cid = lax.axis_index("c")
    sid = lax.axis_index("s")
    w = cid * NS + sid

    zeros16 = jnp.zeros((16,), jnp.float32)
    ones16 = jnp.ones((16,), jnp.float32)

    def zstage(k, carry):
        tmp_v[pl.ds(k * 16, 16)] = zeros16
        return carry

    lax.fori_loop(0, RPT // 16, zstage, 0)

    ones_v[pl.ds(0, 16)] = ones16
    ones_v[pl.ds(16, 16)] = ones16
    ones_v[pl.ds(CK - 16, 16)] = ones16

    pltpu.sync_copy(tmp_v, acc_sh.at[pl.ds(sid * RPT, RPT)])
    pltpu.sync_copy(dst_hbm.at[pl.ds(w * EPW, EPW)], didx_v)
    plsc.subcore_barrier()

    def wave(g, carry):
        def fire(i, c):
            pltpu.async_copy(
                ones_v, acc_sh.at[didx_v.at[pl.ds((g * 25 + i) * CK, CK)]],
                sem, add=True)
            return c

        lax.fori_loop(0, 25, fire, 0)

        def drain(i, c):
            pltpu.make_async_copy(
                dst_hbm.at[pl.ds(0, CK)], didx_v.at[pl.ds(0, CK)], sem).wait()
            return c

        lax.fori_loop(0, 25, drain, 0)
        return carry

    lax.fori_loop(0, NCH // 25, wave, 0)
    plsc.subcore_barrier()

    pltpu.sync_copy(acc_sh.at[pl.ds(sid * RPT, RPT)], tmp_v)
    pltpu.sync_copy(tmp_v, deg_out.at[cid, pl.ds(sid * RPT, RPT)])


@functools.partial(
    pl.kernel,
    out_type=jax.ShapeDtypeStruct((NC, NP, D), jnp.float32),
    mesh=_MESH,
    scratch_types=[
        pltpu.VMEM_SHARED((NP, D), jnp.float32),
        pltpu.VMEM((EPW,), jnp.int32),
        pltpu.VMEM((EPW,), jnp.int32),
        [pltpu.VMEM((CK, D), jnp.float32) for _ in range(5)],
        pltpu.SemaphoreType.DMA((5,)),
        pltpu.SemaphoreType.DMA((5,)),
    ],
)
def _agg_kernel(g_hbm, src_hbm, dst_hbm, part_out,
                acc_sh, sidx_v, didx_v, rows, gsem, ssem):
    NR = 5
    NG = NCH // NR
    cid = lax.axis_index("c")
    sid = lax.axis_index("s")
    w = cid * NS + sid

    zeros16 = jnp.zeros((16,), jnp.float32)

    def ztmp(k, carry):
        rows[0][k // 8, pl.ds((k % 8) * 16, 16)] = zeros16
        return carry

    lax.fori_loop(0, CK * (D // 16), ztmp, 0)

    def zacc(k, carry):
        pltpu.async_copy(rows[0], acc_sh.at[pl.ds(sid * RPT + k * CK, CK), :],
                         gsem.at[0])
        return carry

    lax.fori_loop(0, NDMP, zacc, 0)
    pltpu.sync_copy(src_hbm.at[pl.ds(w * EPW, EPW)], sidx_v)
    pltpu.sync_copy(dst_hbm.at[pl.ds(w * EPW, EPW)], didx_v)

    def zdrain(k, carry):
        pltpu.make_async_copy(g_hbm.at[pl.ds(0, CK)], rows[0], gsem.at[0]).wait()
        return carry

    lax.fori_loop(0, NDMP, zdrain, 0)
    plsc.subcore_barrier()

    def emit_group(g, drain_scatter):
        gcps = []
        for b in range(NR):
            i = g * NR + b
            if drain_scatter:
                pltpu.make_async_copy(g_hbm.at[pl.ds(0, CK)], rows[b],
                                      ssem.at[b]).wait()
            gcps.append(pltpu.async_copy(
                g_hbm.at[sidx_v.at[pl.ds(i * CK, CK)]], rows[b], gsem.at[b]))
        for b in range(NR):
            i = g * NR + b
            gcps[b].wait()
            pltpu.async_copy(rows[b], acc_sh.at[didx_v.at[pl.ds(i * CK, CK)]],
                             ssem.at[b], add=True)

    emit_group(0, False)

    def group(g, carry):
        emit_group(g, True)
        return carry

    lax.fori_loop(1, NG, group, 0)
    for b in range(NR):
        pltpu.make_async_copy(g_hbm.at[pl.ds(0, CK)], rows[b], ssem.at[b]).wait()
    plsc.subcore_barrier()

    def demit(k, drain):
        for b in range(4):
            r0 = sid * RPT + (k * 4 + b) * CK
            if drain:
                pltpu.make_async_copy(g_hbm.at[pl.ds(0, CK)], rows[b],
                                      ssem.at[b]).wait()
            pltpu.async_copy(acc_sh.at[pl.ds(r0, CK), :], rows[b], gsem.at[b])
        for b in range(4):
            r0 = sid * RPT + (k * 4 + b) * CK
            pltpu.make_async_copy(g_hbm.at[pl.ds(0, CK)], rows[b],
                                  gsem.at[b]).wait()
            pltpu.async_copy(rows[b], part_out.at[cid, pl.ds(r0, CK), :],
                             ssem.at[b])

    demit(0, False)

    def dgroup(k, carry):
        demit(k, True)
        return carry

    lax.fori_loop(1, NDMP // 4, dgroup, 0)
    for b in range(4):
        pltpu.make_async_copy(g_hbm.at[pl.ds(0, CK)], rows[b], ssem.at[b]).wait()


BN = 2000


def _tca_body(degp, x, w1, g1, dinv):
    deg = degp[0] + degp[1] + 1.0
    di = lax.rsqrt(deg)
    h = jnp.dot(x[...], w1[...], preferred_element_type=jnp.float32)
    g1[...] = h * di
    dinv[...] = di


_tca = pl.pallas_call(
    _tca_body,
    grid=(N // BN,),
    in_specs=[
        pl.BlockSpec((NC, BN, 1), lambda i: (0, i, 0)),
        pl.BlockSpec((BN, D), lambda i: (i, 0)),
        pl.BlockSpec((D, D), lambda i: (0, 0)),
    ],
    out_specs=[
        pl.BlockSpec((BN, D), lambda i: (i, 0)),
        pl.BlockSpec((BN, 1), lambda i: (i, 0)),
    ],
    out_shape=[
        jax.ShapeDtypeStruct((N, D), jnp.float32),
        jax.ShapeDtypeStruct((N, 1), jnp.float32),
    ],
)


def _tcb_body(p, g1, dinv, b1, w2, g2):
    agg = p[0] + p[1] + g1[...]
    di = dinv[...]
    out1 = jnp.maximum(agg * di + b1[...], 0.0)
    h2 = jnp.dot(out1, w2[...], preferred_element_type=jnp.float32)
    g2[...] = h2 * di


_tcb = pl.pallas_call(
    _tcb_body,
    grid=(N // BN,),
    in_specs=[
        pl.BlockSpec((NC, BN, D), lambda i: (0, i, 0)),
        pl.BlockSpec((BN, D), lambda i: (i, 0)),
        pl.BlockSpec((BN, 1), lambda i: (i, 0)),
        pl.BlockSpec((1, D), lambda i: (0, 0)),
        pl.BlockSpec((D, D), lambda i: (0, 0)),
    ],
    out_specs=pl.BlockSpec((BN, D), lambda i: (i, 0)),
    out_shape=jax.ShapeDtypeStruct((N, D), jnp.float32),
)


def _tcc_body(p, g2, dinv, b2, out):
    out[...] = (p[0] + p[1] + g2[...]) * dinv[...] + b2[...]


_tcc = pl.pallas_call(
    _tcc_body,
    grid=(N // BN,),
    in_specs=[
        pl.BlockSpec((NC, BN, D), lambda i: (0, i, 0)),
        pl.BlockSpec((BN, D), lambda i: (i, 0)),
        pl.BlockSpec((BN, 1), lambda i: (i, 0)),
        pl.BlockSpec((1, D), lambda i: (0, 0)),
    ],
    out_specs=pl.BlockSpec((BN, D), lambda i: (i, 0)),
    out_shape=jax.ShapeDtypeStruct((N, D), jnp.float32),
)


def kernel(x, edge_index, W1, b1, W2, b2):
    ei = edge_index.astype(jnp.int32)
    src = ei[0]
    dst = ei[1]
    b1r = b1.reshape(1, D)
    b2r = b2.reshape(1, D)

    degp = _deg_kernel(dst).reshape(NC, NP, 1)
    g1, dinv = _tca(degp, x, W1)
    part1 = _agg_kernel(g1, src, dst)
    g2 = _tcb(part1, g1, dinv, b1r, W2)
    part2 = _agg_kernel(g2, src, dst)
    out = _tcc(part2, g2, dinv, b2r)
    return out
